# baseline (speedup 1.0000x reference)
import functools

import jax
import jax.numpy as jnp
from jax import lax
from jax.experimental import pallas as pl
from jax.experimental.pallas import tpu as pltpu
from jax.experimental.pallas import tpu_sc as plsc

_NUM_FIELDS = 26
_FIELD_DIM = 40000
_BUCKET = 100000
_D = 64
_GK = 0.02
_QPR = 11
_B = 4096
_N = _B * _NUM_FIELDS

_NC = 2
_NS = 16
_NW = _NC * _NS
_L = 16
_ROWS_PER_W = _N // _NW
_CHUNK = 128
_NCHUNKS = _ROWS_PER_W // _CHUNK


def _prune(v, s):
    t = _GK / (1.0 + jnp.exp(-s))
    return jnp.sign(v) * jnp.maximum(jnp.abs(v) - t, 0.0)


def _sc_body(x_hbm, qv_hbm, rv_hbm, qs_hbm, rs_hbm, out_hbm,
             xp_v, idxq_v, idxr_v, qv_v, qs_v, rv_v, rs_v, sem):
    wid = lax.axis_index("s") * _NC + lax.axis_index("c")
    base = wid * _ROWS_PER_W

    def chunk_body(c, carry):
        gbase = base + c * _CHUNK
        pltpu.sync_copy(x_hbm.at[pl.ds(gbase, _CHUNK)], xp_v)

        lane = lax.iota(jnp.int32, _L)

        def idx_body(j, carry2):
            xv = xp_v[pl.ds(j * _L, _L)]
            pv = gbase + j * _L + lane
            col = lax.rem(pv, _NUM_FIELDS)
            xn = xv + col * _FIELD_DIM
            idxq_v[pl.ds(j * _L, _L)] = lax.div(xn, _QPR)
            idxr_v[pl.ds(j * _L, _L)] = lax.rem(xn, _BUCKET)
            return carry2

        lax.fori_loop(0, _CHUNK // _L, idx_body, 0, unroll=2)

        cp0 = pltpu.async_copy(qv_hbm.at[idxq_v], qv_v, sem)
        cp1 = pltpu.async_copy(qs_hbm.at[idxq_v], qs_v, sem)
        cp2 = pltpu.async_copy(rv_hbm.at[idxr_v], rv_v, sem)
        cp3 = pltpu.async_copy(rs_hbm.at[idxr_v], rs_v, sem)
        cp0.wait()
        cp1.wait()
        cp2.wait()
        cp3.wait()

        def row_body(i, carry2):
            for k in range(_D // _L):
                sl = pl.ds(k * _L, _L)
                q = _prune(qv_v[i, sl], qs_v[i, sl])
                r = _prune(rv_v[i, sl], rs_v[i, sl])
                qv_v[i, sl] = q + r
            return carry2

        lax.fori_loop(0, _CHUNK, row_body, 0)

        pltpu.sync_copy(qv_v, out_hbm.at[pl.ds(gbase, _CHUNK)])
        return carry

    lax.fori_loop(0, _NCHUNKS, chunk_body, 0)


_mesh = plsc.VectorSubcoreMesh(core_axis_name="c", subcore_axis_name="s")

_ce_kernel = functools.partial(
    pl.kernel,
    out_type=jax.ShapeDtypeStruct((_N, _D), jnp.float32),
    mesh=_mesh,
    scratch_types=[
        pltpu.VMEM((_CHUNK,), jnp.int32),
        pltpu.VMEM((_CHUNK,), jnp.int32),
        pltpu.VMEM((_CHUNK,), jnp.int32),
        pltpu.VMEM((_CHUNK, _D), jnp.float32),
        pltpu.VMEM((_CHUNK, _D), jnp.float32),
        pltpu.VMEM((_CHUNK, _D), jnp.float32),
        pltpu.VMEM((_CHUNK, _D), jnp.float32),
        pltpu.SemaphoreType.DMA,
    ],
    compiler_params=pltpu.CompilerParams(use_tc_tiling_on_sc=False),
)(_sc_body)


def kernel(x, Q_v, R_v, Q_s, R_s):
    x_flat = x.reshape(_N)
    out = _ce_kernel(x_flat, Q_v, R_v, Q_s, R_s)
    return out.reshape(_B, _NUM_FIELDS, _D)
